# baseline (speedup 1.0000x reference)
import jax
import jax.numpy as jnp
from jax import lax
from jax.experimental import pallas as pl
from jax.experimental.pallas import tpu as pltpu
from jax.experimental.pallas import tpu_sc as plsc

VOCAB = 1000000
DIM = 64
SEQ = 200
BATCH = 4096

NC, NS = 2, 16
NW = NC * NS
BBLK = BATCH // NW
SPLITS = ((0, 104), (104, 96))


def _make_body(s0, ns):
    def _body(tok_hbm, bias_hbm, table_hbm, out_hbm, tok_v, bias_v,
              gbuf0, gbuf1, obuf0, obuf1, gsem0, gsem1, osem0, osem1):
        wid = lax.axis_index("s") * NC + lax.axis_index("c")
        pltpu.sync_copy(tok_hbm.at[wid, pl.ds(s0, ns)], tok_v)
        pltpu.sync_copy(bias_hbm.at[pl.ds(wid * BBLK, BBLK)], bias_v)
        gbufs = (gbuf0, gbuf1)
        obufs = (obuf0, obuf1)
        gsems = (gsem0, gsem1)
        osems = (osem0, osem1)

        def issue_gather(s, b):
            pltpu.async_copy(table_hbm.at[tok_v.at[s]], gbufs[b], gsems[b])

        def wait_gather(s, b):
            pltpu.make_async_copy(table_hbm.at[tok_v.at[s]], gbufs[b],
                                  gsems[b]).wait()

        def issue_write(s, b):
            pltpu.async_copy(obufs[b],
                             out_hbm.at[s, pl.ds(wid * BBLK, BBLK)],
                             osems[b])

        def wait_write(s, b):
            pltpu.make_async_copy(obufs[b],
                                  out_hbm.at[s, pl.ds(wid * BBLK, BBLK)],
                                  osems[b]).wait()

        def add_bias(b):
            gb, ob = gbufs[b], obufs[b]

            def jloop(j, _):
                for k in range(DIM // 16):
                    ob[j, pl.ds(k * 16, 16)] = (
                        gb[j, pl.ds(k * 16, 16)]
                        + bias_v[j, pl.ds(k * 16, 16)])
                return 0

            lax.fori_loop(0, BBLK, jloop, 0)

        issue_gather(0, 0)

        def outer(cc, _):
            for b in range(2):
                s = cc * 2 + b

                @pl.when(s >= 2)
                def _():
                    wait_write(s - 2, b)

                @pl.when(s + 1 < ns)
                def _():
                    issue_gather(s + 1, 1 - b)

                wait_gather(s, b)
                add_bias(b)
                issue_write(s, b)
            return 0

        lax.fori_loop(0, ns // 2, outer, 0)
        wait_write(ns - 2, 0)
        wait_write(ns - 1, 1)

    return _body


@jax.jit
def _run(tok_blocked, bias, table_padded):
    mesh = plsc.VectorSubcoreMesh(core_axis_name="c", subcore_axis_name="s")
    outs = []
    for h, (s0, ns) in enumerate(SPLITS):
        f = pl.kernel(
            _make_body(s0, ns),
            out_type=jax.ShapeDtypeStruct((ns, BATCH, DIM), jnp.float32),
            mesh=mesh,
            scratch_types=[
                pltpu.VMEM((ns, BBLK), jnp.int32),
                pltpu.VMEM((BBLK, DIM), jnp.float32),
                pltpu.VMEM((BBLK, 128), jnp.float32),
                pltpu.VMEM((BBLK, 128), jnp.float32),
                pltpu.VMEM((BBLK, DIM), jnp.float32),
                pltpu.VMEM((BBLK, DIM), jnp.float32),
                pltpu.SemaphoreType.DMA,
                pltpu.SemaphoreType.DMA,
                pltpu.SemaphoreType.DMA,
                pltpu.SemaphoreType.DMA,
            ],
            compiler_params=pltpu.CompilerParams(use_tc_tiling_on_sc=True),
            name=f"gather_half{h}",
        )
        outs.append(f(tok_blocked, bias, table_padded))
    return jnp.concatenate(outs, axis=0)


def kernel(tokens, table, condition_bias):
    tok_blocked = (tokens.astype(jnp.int32)
                   .reshape(SEQ, NW, BBLK)
                   .transpose(1, 0, 2))
    table_padded = jnp.pad(table, ((0, 0), (0, 128 - DIM)))
    return _run(tok_blocked, condition_bias, table_padded)


# speedup vs baseline: 1.1201x; 1.1201x over previous
import jax
import jax.numpy as jnp
from jax import lax
from jax.experimental import pallas as pl
from jax.experimental.pallas import tpu as pltpu
from jax.experimental.pallas import tpu_sc as plsc

VOCAB = 1000000
DIM = 64
SEQ = 200
BATCH = 4096

NC, NS = 2, 16
NW = NC * NS
BBLK = BATCH // NW


def _body(tok_hbm, bias_hbm, table_hbm, out_hbm, tok_v, bias_v,
          gbuf0, gbuf1, obuf0, obuf1, gsem0, gsem1, osem0, osem1):
    wid = lax.axis_index("s") * NC + lax.axis_index("c")
    pltpu.sync_copy(tok_hbm.at[wid], tok_v)
    pltpu.sync_copy(bias_hbm.at[pl.ds(wid * BBLK, BBLK)], bias_v)
    gbufs = (gbuf0, gbuf1)
    obufs = (obuf0, obuf1)
    gsems = (gsem0, gsem1)
    osems = (osem0, osem1)

    def issue_gather(s, b):
        pltpu.async_copy(table_hbm.at[tok_v.at[s]], gbufs[b], gsems[b])

    def wait_gather(s, b):
        pltpu.make_async_copy(table_hbm.at[tok_v.at[s]], gbufs[b],
                              gsems[b]).wait()

    def issue_write(s, b):
        pltpu.async_copy(obufs[b],
                         out_hbm.at[s, pl.ds(wid * BBLK, BBLK)], osems[b])

    def wait_write(s, b):
        pltpu.make_async_copy(obufs[b],
                              out_hbm.at[s, pl.ds(wid * BBLK, BBLK)],
                              osems[b]).wait()

    def add_bias(b):
        gb, ob = gbufs[b], obufs[b]

        def jloop(j, _):
            for k in range(DIM // 16):
                ob[j, pl.ds(k * 16, 16)] = (
                    gb[j, pl.ds(k * 16, 16)] + bias_v[j, pl.ds(k * 16, 16)])
            return 0

        lax.fori_loop(0, BBLK, jloop, 0)

    issue_gather(0, 0)

    def outer(cc, _):
        for b in range(2):
            s = cc * 2 + b

            @pl.when(s >= 2)
            def _():
                wait_write(s - 2, b)

            @pl.when(s + 1 < SEQ)
            def _():
                issue_gather(s + 1, 1 - b)

            wait_gather(s, b)
            add_bias(b)
            issue_write(s, b)
        return 0

    lax.fori_loop(0, SEQ // 2, outer, 0)
    wait_write(SEQ - 2, 0)
    wait_write(SEQ - 1, 1)


@jax.jit
def _run(tok_blocked, bias, table_padded):
    mesh = plsc.VectorSubcoreMesh(core_axis_name="c", subcore_axis_name="s")
    f = pl.kernel(
        _body,
        out_type=jax.ShapeDtypeStruct((SEQ, BATCH, DIM), jnp.float32),
        mesh=mesh,
        scratch_types=[
            pltpu.VMEM((SEQ, BBLK), jnp.int32),
            pltpu.VMEM((BBLK, DIM), jnp.float32),
            pltpu.VMEM((BBLK, 128), jnp.float32),
            pltpu.VMEM((BBLK, 128), jnp.float32),
            pltpu.VMEM((BBLK, DIM), jnp.float32),
            pltpu.VMEM((BBLK, DIM), jnp.float32),
            pltpu.SemaphoreType.DMA,
            pltpu.SemaphoreType.DMA,
            pltpu.SemaphoreType.DMA,
            pltpu.SemaphoreType.DMA,
        ],
        compiler_params=pltpu.CompilerParams(use_tc_tiling_on_sc=True),
    )
    return f(tok_blocked, bias, table_padded)


def kernel(tokens, table, condition_bias):
    tok_blocked = (tokens.astype(jnp.int32)
                   .reshape(SEQ, NW, BBLK)
                   .transpose(1, 0, 2))
    table_padded = jnp.pad(table, ((0, 0), (0, 128 - DIM)))
    return _run(tok_blocked, condition_bias, table_padded)
